# baseline (device time: 13080 ns/iter reference)
import jax
import jax.numpy as jnp
from jax import lax
from jax.experimental import pallas as pl
from jax.experimental.pallas import tpu as pltpu

N_DEV = 4


def kernel(x, Wq, K_ext, V_ext, Wo):
    B, Sql, E = x.shape
    _, Skl, Hq, Dh = K_ext.shape
    HD = Hq * Dh
    Skv = N_DEV * Skl

    def body(x_ref, wq_ref, k_ref, v_ref, wo_ref, out_ref,
             kfull, vfull, ksc, vsc, kst, vst,
             ksend, krecv, vsend, vrecv, ssend, srecv, insem):
        my = lax.axis_index("i")
        left = lax.rem(my + (N_DEV - 1), N_DEV)
        right = lax.rem(my + 1, N_DEV)
        diag = lax.rem(my + 2, N_DEV)
        peers = (left, right, diag)

        cp_k = pltpu.make_async_copy(k_ref, kst, insem.at[0])
        cp_v = pltpu.make_async_copy(v_ref, vst, insem.at[1])
        cp_k.start()
        cp_v.start()

        barrier = pltpu.get_barrier_semaphore()
        for nbr in peers:
            pltpu.semaphore_signal(
                barrier, inc=1,
                device_id=(nbr,), device_id_type=pl.DeviceIdType.MESH,
            )

        def quantize(st_ref, qfull, qsc, cp):
            cp.wait()
            val = st_ref[...].reshape(B, Skl, HD)
            amax = jnp.maximum(jnp.max(jnp.abs(val), axis=-1), 1e-20)
            scale = amax * (1.0 / 127.0)
            qfull[:, pl.ds(my * Skl, Skl), :] = jnp.round(
                val / scale[:, :, None]).astype(jnp.int8)
            qsc[:, pl.ds(my * Skl, Skl)] = scale

        quantize(kst, kfull, ksc, cp_k)
        pltpu.semaphore_wait(barrier, len(peers))

        sends = []

        def send_tensor(i, buf, ssm, rsm, scbuf):
            for j, dest in enumerate(peers):
                r = pltpu.make_async_remote_copy(
                    src_ref=buf.at[:, pl.ds(my * Skl, Skl), :],
                    dst_ref=buf.at[:, pl.ds(my * Skl, Skl), :],
                    send_sem=ssm.at[j], recv_sem=rsm.at[j],
                    device_id=(dest,), device_id_type=pl.DeviceIdType.MESH,
                )
                r.start()
                sends.append(r)
                r = pltpu.make_async_remote_copy(
                    src_ref=scbuf.at[:, pl.ds(my * Skl, Skl)],
                    dst_ref=scbuf.at[:, pl.ds(my * Skl, Skl)],
                    send_sem=ssend.at[i, j], recv_sem=srecv.at[i, j],
                    device_id=(dest,), device_id_type=pl.DeviceIdType.MESH,
                )
                r.start()
                sends.append(r)

        send_tensor(0, kfull, ksend, krecv, ksc)
        quantize(vst, vfull, vsc, cp_v)
        send_tensor(1, vfull, vsend, vrecv, vsc)

        wq = wq_ref[...].astype(jnp.bfloat16)
        wo = wo_ref[...].astype(jnp.bfloat16)

        q = [
            (jnp.dot(x_ref[b].astype(jnp.bfloat16), wq,
                     preferred_element_type=jnp.float32)
             * 0.125).astype(jnp.bfloat16)
            for b in range(B)
        ]

        qrow = lax.broadcasted_iota(jnp.int32, (Sql, Skl), 0) + my * Sql
        kcol = lax.broadcasted_iota(jnp.int32, (Sql, Skl), 1)

        acc = [[jnp.zeros((Sql, Dh), jnp.float32) for _ in range(Hq)]
               for _ in range(B)]
        lsum = [[jnp.zeros((Sql, 1), jnp.float32) for _ in range(Hq)]
                for _ in range(B)]

        def process_block(origin):
            ki = kcol + origin * Skl
            mask = (jnp.abs(qrow - ki) <= 128) | (ki < 32) | (qrow < 32)
            for b in range(B):
                kb = kfull[b, pl.ds(origin * Skl, Skl), :]
                vb = vfull[b, pl.ds(origin * Skl, Skl), :]
                ks = ksc[b, pl.ds(origin * Skl, Skl)]
                vs = vsc[b, pl.ds(origin * Skl, Skl)]
                for h in range(Hq):
                    qh = q[b][:, h * Dh:(h + 1) * Dh]
                    kh = kb[:, h * Dh:(h + 1) * Dh].astype(jnp.bfloat16)
                    s = lax.dot_general(
                        qh, kh, (((1,), (1,)), ((), ())),
                        preferred_element_type=jnp.float32,
                    ) * ks[None, :]
                    p = jnp.exp(jnp.where(mask, s, -1e9))
                    lsum[b][h] = lsum[b][h] + jnp.sum(p, axis=-1,
                                                      keepdims=True)
                    acc[b][h] = acc[b][h] + jnp.dot(
                        (p * vs[None, :]).astype(jnp.bfloat16),
                        vb[:, h * Dh:(h + 1) * Dh].astype(jnp.bfloat16),
                        preferred_element_type=jnp.float32,
                    )

        process_block(my)
        for j, origin in enumerate((right, left, diag)):
            pltpu.make_async_remote_copy(
                src_ref=kfull.at[:, pl.ds(origin * Skl, Skl), :],
                dst_ref=kfull.at[:, pl.ds(origin * Skl, Skl), :],
                send_sem=ksend.at[j], recv_sem=krecv.at[j],
                device_id=(origin,), device_id_type=pl.DeviceIdType.MESH,
            ).wait_recv()
            pltpu.make_async_remote_copy(
                src_ref=vfull.at[:, pl.ds(origin * Skl, Skl), :],
                dst_ref=vfull.at[:, pl.ds(origin * Skl, Skl), :],
                send_sem=vsend.at[j], recv_sem=vrecv.at[j],
                device_id=(origin,), device_id_type=pl.DeviceIdType.MESH,
            ).wait_recv()
            for i, buf in enumerate((ksc, vsc)):
                pltpu.make_async_remote_copy(
                    src_ref=buf.at[:, pl.ds(origin * Skl, Skl)],
                    dst_ref=buf.at[:, pl.ds(origin * Skl, Skl)],
                    send_sem=ssend.at[i, j], recv_sem=srecv.at[i, j],
                    device_id=(origin,), device_id_type=pl.DeviceIdType.MESH,
                ).wait_recv()
            process_block(origin)

        for b in range(B):
            ctx = jnp.concatenate(
                [(acc[b][h] / lsum[b][h]).astype(jnp.bfloat16)
                 for h in range(Hq)],
                axis=1,
            )
            out_ref[b] = jnp.dot(ctx, wo,
                                 preferred_element_type=jnp.float32)

        for r in sends:
            r.wait_send()

    return pl.pallas_call(
        body,
        out_shape=jax.ShapeDtypeStruct((B, Sql, E), jnp.float32),
        in_specs=[
            pl.BlockSpec(memory_space=pltpu.VMEM),
            pl.BlockSpec(memory_space=pltpu.VMEM),
            pl.BlockSpec(memory_space=pl.ANY),
            pl.BlockSpec(memory_space=pl.ANY),
            pl.BlockSpec(memory_space=pltpu.VMEM),
        ],
        out_specs=pl.BlockSpec(memory_space=pltpu.VMEM),
        scratch_shapes=[
            pltpu.VMEM((B, Skv, HD), jnp.int8),
            pltpu.VMEM((B, Skv, HD), jnp.int8),
            pltpu.VMEM((B, Skv), jnp.float32),
            pltpu.VMEM((B, Skv), jnp.float32),
            pltpu.VMEM((B, Skl, Hq, Dh), jnp.float32),
            pltpu.VMEM((B, Skl, Hq, Dh), jnp.float32),
            pltpu.SemaphoreType.DMA((3,)),
            pltpu.SemaphoreType.DMA((3,)),
            pltpu.SemaphoreType.DMA((3,)),
            pltpu.SemaphoreType.DMA((3,)),
            pltpu.SemaphoreType.DMA((2, 3)),
            pltpu.SemaphoreType.DMA((2, 3)),
            pltpu.SemaphoreType.DMA((2,)),
        ],
        compiler_params=pltpu.CompilerParams(collective_id=0),
    )(x, Wq, K_ext, V_ext, Wo)


# device time: 12929 ns/iter; 1.0117x vs baseline; 1.0117x over previous
import jax
import jax.numpy as jnp
from jax import lax
from jax.experimental import pallas as pl
from jax.experimental.pallas import tpu as pltpu

N_DEV = 4


def kernel(x, Wq, K_ext, V_ext, Wo):
    B, Sql, E = x.shape
    _, Skl, Hq, Dh = K_ext.shape
    HD = Hq * Dh
    Skv = N_DEV * Skl

    def body(x_ref, wq_ref, k_ref, v_ref, wo_ref, out_ref,
             kfull, vfull, ksc, vsc, kst, vst,
             ksend, krecv, vsend, vrecv, ssend, srecv, insem):
        my = lax.axis_index("i")
        left = lax.rem(my + (N_DEV - 1), N_DEV)
        right = lax.rem(my + 1, N_DEV)
        diag = lax.rem(my + 2, N_DEV)
        peers = (left, right, diag)

        cp_k = pltpu.make_async_copy(k_ref, kst, insem.at[0])
        cp_v = pltpu.make_async_copy(v_ref, vst, insem.at[1])
        cp_k.start()
        cp_v.start()

        barrier = pltpu.get_barrier_semaphore()
        for nbr in peers:
            pltpu.semaphore_signal(
                barrier, inc=1,
                device_id=(nbr,), device_id_type=pl.DeviceIdType.MESH,
            )

        def quantize(st_ref, qfull, qsc, cp):
            cp.wait()
            val = st_ref[...].reshape(B, Skl, HD)
            amax = jnp.maximum(jnp.max(jnp.abs(val), axis=-1), 1e-20)
            scale = amax * (1.0 / 127.0)
            qfull[:, pl.ds(my * Skl, Skl), :] = jnp.round(
                val / scale[:, :, None]).astype(jnp.int8)
            qsc[:, pl.ds(my * Skl, Skl)] = scale

        quantize(kst, kfull, ksc, cp_k)
        quantize(vst, vfull, vsc, cp_v)

        pltpu.semaphore_wait(barrier, len(peers))

        sends = []
        for j, dest in enumerate(peers):
            for buf, ssm, rsm in ((kfull, ksend, krecv),
                                  (vfull, vsend, vrecv)):
                r = pltpu.make_async_remote_copy(
                    src_ref=buf.at[:, pl.ds(my * Skl, Skl), :],
                    dst_ref=buf.at[:, pl.ds(my * Skl, Skl), :],
                    send_sem=ssm.at[j], recv_sem=rsm.at[j],
                    device_id=(dest,), device_id_type=pl.DeviceIdType.MESH,
                )
                r.start()
                sends.append(r)
            for i, buf in enumerate((ksc, vsc)):
                r = pltpu.make_async_remote_copy(
                    src_ref=buf.at[:, pl.ds(my * Skl, Skl)],
                    dst_ref=buf.at[:, pl.ds(my * Skl, Skl)],
                    send_sem=ssend.at[i, j], recv_sem=srecv.at[i, j],
                    device_id=(dest,), device_id_type=pl.DeviceIdType.MESH,
                )
                r.start()
                sends.append(r)

        wq = wq_ref[...].astype(jnp.bfloat16)
        wo = wo_ref[...].astype(jnp.bfloat16)

        q = [
            (jnp.dot(x_ref[b].astype(jnp.bfloat16), wq,
                     preferred_element_type=jnp.float32)
             * 0.125).astype(jnp.bfloat16)
            for b in range(B)
        ]

        qrow = lax.broadcasted_iota(jnp.int32, (Sql, Skl), 0) + my * Sql
        kcol = lax.broadcasted_iota(jnp.int32, (Sql, Skl), 1)

        acc = [[jnp.zeros((Sql, Dh), jnp.float32) for _ in range(Hq)]
               for _ in range(B)]
        lsum = [[jnp.zeros((Sql, 1), jnp.float32) for _ in range(Hq)]
                for _ in range(B)]

        def process_block(origin):
            ki = kcol + origin * Skl
            mask = (jnp.abs(qrow - ki) <= 128) | (ki < 32) | (qrow < 32)
            for b in range(B):
                kb = kfull[b, pl.ds(origin * Skl, Skl), :]
                vb = vfull[b, pl.ds(origin * Skl, Skl), :]
                ks = ksc[b, pl.ds(origin * Skl, Skl)]
                vs = vsc[b, pl.ds(origin * Skl, Skl)]
                for h in range(Hq):
                    qh = q[b][:, h * Dh:(h + 1) * Dh]
                    kh = kb[:, h * Dh:(h + 1) * Dh].astype(jnp.bfloat16)
                    s = lax.dot_general(
                        qh, kh, (((1,), (1,)), ((), ())),
                        preferred_element_type=jnp.float32,
                    ) * ks[None, :]
                    p = jnp.exp(jnp.where(mask, s, -1e9))
                    lsum[b][h] = lsum[b][h] + jnp.sum(p, axis=-1,
                                                      keepdims=True)
                    acc[b][h] = acc[b][h] + jnp.dot(
                        (p * vs[None, :]).astype(jnp.bfloat16),
                        vb[:, h * Dh:(h + 1) * Dh].astype(jnp.bfloat16),
                        preferred_element_type=jnp.float32,
                    )

        process_block(my)
        for j, origin in enumerate((right, left, diag)):
            pltpu.make_async_remote_copy(
                src_ref=kfull.at[:, pl.ds(origin * Skl, Skl), :],
                dst_ref=kfull.at[:, pl.ds(origin * Skl, Skl), :],
                send_sem=ksend.at[j], recv_sem=krecv.at[j],
                device_id=(origin,), device_id_type=pl.DeviceIdType.MESH,
            ).wait_recv()
            pltpu.make_async_remote_copy(
                src_ref=vfull.at[:, pl.ds(origin * Skl, Skl), :],
                dst_ref=vfull.at[:, pl.ds(origin * Skl, Skl), :],
                send_sem=vsend.at[j], recv_sem=vrecv.at[j],
                device_id=(origin,), device_id_type=pl.DeviceIdType.MESH,
            ).wait_recv()
            for i, buf in enumerate((ksc, vsc)):
                pltpu.make_async_remote_copy(
                    src_ref=buf.at[:, pl.ds(origin * Skl, Skl)],
                    dst_ref=buf.at[:, pl.ds(origin * Skl, Skl)],
                    send_sem=ssend.at[i, j], recv_sem=srecv.at[i, j],
                    device_id=(origin,), device_id_type=pl.DeviceIdType.MESH,
                ).wait_recv()
            process_block(origin)

        for b in range(B):
            ctx = jnp.concatenate(
                [(acc[b][h] / lsum[b][h]).astype(jnp.bfloat16)
                 for h in range(Hq)],
                axis=1,
            )
            out_ref[b] = jnp.dot(ctx, wo,
                                 preferred_element_type=jnp.float32)

        for r in sends:
            r.wait_send()

    return pl.pallas_call(
        body,
        out_shape=jax.ShapeDtypeStruct((B, Sql, E), jnp.float32),
        in_specs=[
            pl.BlockSpec(memory_space=pltpu.VMEM),
            pl.BlockSpec(memory_space=pltpu.VMEM),
            pl.BlockSpec(memory_space=pl.ANY),
            pl.BlockSpec(memory_space=pl.ANY),
            pl.BlockSpec(memory_space=pltpu.VMEM),
        ],
        out_specs=pl.BlockSpec(memory_space=pltpu.VMEM),
        scratch_shapes=[
            pltpu.VMEM((B, Skv, HD), jnp.int8),
            pltpu.VMEM((B, Skv, HD), jnp.int8),
            pltpu.VMEM((B, Skv), jnp.float32),
            pltpu.VMEM((B, Skv), jnp.float32),
            pltpu.VMEM((B, Skl, Hq, Dh), jnp.float32),
            pltpu.VMEM((B, Skl, Hq, Dh), jnp.float32),
            pltpu.SemaphoreType.DMA((3,)),
            pltpu.SemaphoreType.DMA((3,)),
            pltpu.SemaphoreType.DMA((3,)),
            pltpu.SemaphoreType.DMA((3,)),
            pltpu.SemaphoreType.DMA((2, 3)),
            pltpu.SemaphoreType.DMA((2, 3)),
            pltpu.SemaphoreType.DMA((2,)),
        ],
        compiler_params=pltpu.CompilerParams(collective_id=0),
    )(x, Wq, K_ext, V_ext, Wo)
